# Initial kernel scaffold; baseline (speedup 1.0000x reference)
import jax
import jax.numpy as jnp
from jax.experimental import pallas as pl


def kernel(x1, batch1, x2, batch2, W):
    raise NotImplementedError("write your pallas kernel here")


# speedup vs baseline: 1.4632x; 1.4632x over previous
import jax
import jax.numpy as jnp
from jax import lax
from jax.experimental import pallas as pl
from jax.experimental.pallas import tpu as pltpu
from jax.experimental.pallas import tpu_sc as plsc

N = 100000
D = 128
G = 256
EPS = 1e-8

NC = 2
NS = 16
NW = NC * NS
VPR = D // 16
NGRP = 6400
NP = NGRP * 16
GPW = NGRP // NW
CHUNK_G = 8
NCHUNK = GPW // CHUNK_G
GSEG = G + 1
ACC_ROWS = GSEG * VPR
CNT_ROWS = 17


def _segsum_body(x1_hbm, b1_hbm, x2_hbm, b2_hbm,
                 p1_out, c1_out, p2_out, c2_out,
                 xbuf, idbuf, acc, cnt):
    wid = lax.axis_index("s") * NC + lax.axis_index("c")
    g0 = wid * GPW
    lanes = lax.iota(jnp.int32, 16)
    one = jnp.float32(1.0)
    zero = jnp.float32(0.0)

    for x_hbm, b_hbm, p_out, c_out in (
        (x1_hbm, b1_hbm, p1_out, c1_out),
        (x2_hbm, b2_hbm, p2_out, c2_out),
    ):
        def _zero(r, _):
            for j in range(VPR):
                acc[r * VPR + j] = jnp.zeros((16,), jnp.float32)
            return 0

        lax.fori_loop(0, GSEG, _zero, 0, unroll=False)
        for r in range(CNT_ROWS):
            cnt[r] = jnp.zeros((16,), jnp.float32)

        pltpu.sync_copy(b_hbm.at[pl.ds(g0, GPW)], idbuf)

        @pl.loop(0, NCHUNK)
        def _chunk(c):
            row0 = (g0 + c * CHUNK_G) * 16 * VPR
            pltpu.sync_copy(x_hbm.at[pl.ds(row0, CHUNK_G * 16 * VPR)], xbuf)
            for g in range(CHUNK_G):
                idv = idbuf[c * CHUNK_G + g]
                for i in range(16):
                    b = lax.squeeze(lax.slice_in_dim(idv, i, i + 1), (0,))
                    rb = b * VPR
                    ib = (g * 16 + i) * VPR
                    for j in range(VPR):
                        plsc.addupdate(acc.at[rb + j], xbuf[ib + j])
                    onehot = jnp.where(lanes == (b % 16), one, zero)
                    plsc.addupdate(cnt.at[b // 16], onehot)

        pltpu.sync_copy(acc, p_out.at[wid])
        pltpu.sync_copy(cnt, c_out.at[wid])


@jax.jit
def _segsum(x1v, b1v, x2v, b2v):
    mesh = plsc.VectorSubcoreMesh(
        core_axis_name="c", subcore_axis_name="s", num_cores=NC
    )
    part = jax.ShapeDtypeStruct((NW, ACC_ROWS, 16), jnp.float32)
    cntt = jax.ShapeDtypeStruct((NW, CNT_ROWS, 16), jnp.float32)
    return pl.kernel(
        _segsum_body,
        out_type=(part, cntt, part, cntt),
        mesh=mesh,
        compiler_params=pltpu.CompilerParams(use_tc_tiling_on_sc=False),
        scratch_types=[
            pltpu.VMEM((CHUNK_G * 16 * VPR, 16), jnp.float32),
            pltpu.VMEM((GPW, 16), jnp.int32),
            pltpu.VMEM((ACC_ROWS, 16), jnp.float32),
            pltpu.VMEM((CNT_ROWS, 16), jnp.float32),
        ],
    )(x1v, b1v, x2v, b2v)


def _finish_body(p1_ref, c1_ref, p2_ref, c2_ref, w_ref, out_ref):
    w = w_ref[...]

    def pooled(p_ref, c_ref):
        s = jnp.sum(p_ref[...], axis=0)[:G]
        cg = jnp.sum(c_ref[...], axis=0)[:G]
        c = jnp.maximum(cg, 1.0)
        m = s / c[:, None]
        return jnp.dot(m, w, preferred_element_type=jnp.float32)

    p1 = pooled(p1_ref, c1_ref)
    p2 = pooled(p2_ref, c2_ref)
    n1 = jnp.maximum(jnp.sqrt(jnp.sum(p1 * p1, axis=-1)), EPS)
    n2 = jnp.maximum(jnp.sqrt(jnp.sum(p2 * p2, axis=-1)), EPS)
    out_ref[...] = jnp.sum(p1 * p2, axis=-1) / (n1 * n2)


@jax.jit
def _finish(part1, cnt1, part2, cnt2, w):
    return pl.pallas_call(
        _finish_body,
        out_shape=jax.ShapeDtypeStruct((G,), jnp.float32),
    )(part1, cnt1, part2, cnt2, w)


def kernel(x1, batch1, x2, batch2, W):
    padx = jnp.zeros((NP - N, D), jnp.float32)
    padb = jnp.full((NP - N,), G, jnp.int32)
    x1v = jnp.concatenate([x1, padx]).reshape(NP * VPR, 16)
    x2v = jnp.concatenate([x2, padx]).reshape(NP * VPR, 16)
    b1v = jnp.concatenate([batch1.astype(jnp.int32), padb]).reshape(NGRP, 16)
    b2v = jnp.concatenate([batch2.astype(jnp.int32), padb]).reshape(NGRP, 16)
    part1, cnt1, part2, cnt2 = _segsum(x1v, b1v, x2v, b2v)
    return _finish(
        part1.reshape(NW, GSEG, D),
        cnt1.reshape(NW, CNT_ROWS * 16),
        part2.reshape(NW, GSEG, D),
        cnt2.reshape(NW, CNT_ROWS * 16),
        W,
    )


# speedup vs baseline: 3.9468x; 2.6974x over previous
import jax
import jax.numpy as jnp
from jax import lax
from jax.experimental import pallas as pl
from jax.experimental.pallas import tpu as pltpu
from jax.experimental.pallas import tpu_sc as plsc

N = 100000
D = 128
G = 256
EPS = 1e-8

NC = 2
NS = 16
NW = NC * NS
VPR = D // 16
NGRP = N // 16
NGRP_PAD = 6400
GPW = NGRP_PAD // NW
CHUNK_G = 10
GSEG = G + 1
ACC_ROWS = GSEG * VPR
CNT_ROWS = 17


def _extract(vec, i):
    return lax.squeeze(lax.slice_in_dim(vec, i, i + 1), (0,))


def _segsum_body(x1_hbm, b_hbm, x2_hbm, p_out, c_out, xbuf, idbuf, acc, cnt):
    wid = lax.axis_index("s") * NC + lax.axis_index("c")
    g0 = wid * GPW
    nchunk = jnp.minimum(GPW // CHUNK_G, (NGRP - g0) // CHUNK_G)
    lanes = lax.iota(jnp.int32, 16)
    zerov = jnp.zeros((16,), jnp.float32)

    @pl.loop(0, 2)
    def _per_array(a):
        @pl.loop(0, GSEG)
        def _zero(r):
            for j in range(VPR):
                acc[r * VPR + j] = zerov

        for r in range(CNT_ROWS):
            cnt[r] = zerov

        pltpu.sync_copy(b_hbm.at[pl.ds(a * NGRP_PAD + g0, GPW)], idbuf)
        cur0 = _extract(idbuf[0], 0)
        carry0 = (cur0,) + (zerov,) * VPR

        def _chunk(c, carry):
            row0 = (g0 + c * CHUNK_G) * 16 * VPR

            @pl.when(a == 0)
            def _():
                pltpu.sync_copy(
                    x1_hbm.at[pl.ds(row0, CHUNK_G * 16 * VPR)], xbuf
                )

            @pl.when(a == 1)
            def _():
                pltpu.sync_copy(
                    x2_hbm.at[pl.ds(row0, CHUNK_G * 16 * VPR)], xbuf
                )

            def _group(g, carry):
                idv = idbuf[c * CHUNK_G + g]
                base = g * 16 * VPR
                cur = carry[0]
                same = jnp.all(idv == cur)

                def _fast(cur, *regs):
                    new = []
                    for j in range(VPR):
                        s = regs[j]
                        for i in range(16):
                            s = s + xbuf[base + i * VPR + j]
                        new.append(s)
                    oh = jnp.where(lanes == cur % 16, jnp.float32(16.0),
                                   jnp.float32(0.0))
                    plsc.addupdate(cnt.at[cur // 16], oh)
                    return (cur, *new)

                def _slow(cur, *regs):
                    regs = list(regs)
                    for i in range(16):
                        b = _extract(idv, i)

                        def _flush(cur, *regs):
                            for j in range(VPR):
                                plsc.addupdate(acc.at[cur * VPR + j], regs[j])
                            return (b,) + (zerov,) * VPR

                        def _keep(cur, *regs):
                            return (cur, *regs)

                        cur, *regs = lax.cond(b != cur, _flush, _keep,
                                              cur, *regs)
                        for j in range(VPR):
                            regs[j] = regs[j] + xbuf[base + i * VPR + j]
                        oh = jnp.where(lanes == b % 16, jnp.float32(1.0),
                                       jnp.float32(0.0))
                        plsc.addupdate(cnt.at[b // 16], oh)
                    return (cur, *regs)

                return lax.cond(same, _fast, _slow, *carry)

            return lax.fori_loop(0, CHUNK_G, _group, carry)

        cur, *regs = lax.fori_loop(0, nchunk, _chunk, carry0)
        for j in range(VPR):
            plsc.addupdate(acc.at[cur * VPR + j], regs[j])

        pltpu.sync_copy(acc, p_out.at[a, wid])
        pltpu.sync_copy(cnt, c_out.at[a, wid])


@jax.jit
def _segsum(x1v, bv, x2v):
    mesh = plsc.VectorSubcoreMesh(
        core_axis_name="c", subcore_axis_name="s", num_cores=NC
    )
    return pl.kernel(
        _segsum_body,
        out_type=(
            jax.ShapeDtypeStruct((2, NW, ACC_ROWS, 16), jnp.float32),
            jax.ShapeDtypeStruct((2, NW, CNT_ROWS, 16), jnp.float32),
        ),
        mesh=mesh,
        compiler_params=pltpu.CompilerParams(
            use_tc_tiling_on_sc=False, needs_layout_passes=False
        ),
        scratch_types=[
            pltpu.VMEM((CHUNK_G * 16 * VPR, 16), jnp.float32),
            pltpu.VMEM((GPW, 16), jnp.int32),
            pltpu.VMEM((ACC_ROWS, 16), jnp.float32),
            pltpu.VMEM((CNT_ROWS, 16), jnp.float32),
        ],
    )(x1v, bv, x2v)


def _finish_body(p_ref, c_ref, w_ref, out_ref):
    w = w_ref[...]

    def pooled(a):
        s = jnp.sum(p_ref[a], axis=0)[:G]
        cg = jnp.sum(c_ref[a], axis=0)[:G]
        c = jnp.maximum(cg, 1.0)
        m = s / c[:, None]
        return jnp.dot(m, w, preferred_element_type=jnp.float32)

    p1 = pooled(0)
    p2 = pooled(1)
    n1 = jnp.maximum(jnp.sqrt(jnp.sum(p1 * p1, axis=-1)), EPS)
    n2 = jnp.maximum(jnp.sqrt(jnp.sum(p2 * p2, axis=-1)), EPS)
    out_ref[...] = jnp.sum(p1 * p2, axis=-1) / (n1 * n2)


@jax.jit
def _finish(part, cnt, w):
    return pl.pallas_call(
        _finish_body,
        out_shape=jax.ShapeDtypeStruct((G,), jnp.float32),
    )(part, cnt, w)


def kernel(x1, batch1, x2, batch2, W):
    x1v = x1.reshape(N * VPR, 16)
    x2v = x2.reshape(N * VPR, 16)
    padb = jnp.full((NGRP_PAD * 16 - N,), G, jnp.int32)
    bv = jnp.concatenate(
        [batch1.astype(jnp.int32), padb, batch2.astype(jnp.int32), padb]
    ).reshape(2 * NGRP_PAD, 16)
    part, cnt = _segsum(x1v, bv, x2v)
    return _finish(
        part.reshape(2, NW, GSEG, D), cnt.reshape(2, NW, CNT_ROWS * 16), W
    )


# speedup vs baseline: 4.9323x; 1.2497x over previous
import jax
import jax.numpy as jnp
from jax import lax
from jax.experimental import pallas as pl
from jax.experimental.pallas import tpu as pltpu
from jax.experimental.pallas import tpu_sc as plsc

N = 100000
D = 128
G = 256
EPS = 1e-8

NC = 2
NS = 16
NW = NC * NS
VPR = D // 16
NGRP = N // 16
NGRP_PAD = 6400
GPW = NGRP_PAD // NW
CHUNK_G = 10
GSEG = G + 1
ACC_ROWS = GSEG * VPR
CNT_ROWS = 17


def _extract(vec, i):
    return lax.squeeze(lax.slice_in_dim(vec, i, i + 1), (0,))


CROWS = CHUNK_G * 16 * VPR


def _segsum_body(x1_hbm, b_hbm, x2_hbm, p_out, c_out,
                 xbuf, idbuf, acc, cnt, sem):
    wid = lax.axis_index("s") * NC + lax.axis_index("c")
    g0 = wid * GPW
    nchunk = jnp.minimum(GPW // CHUNK_G, (NGRP - g0) // CHUNK_G)
    lanes = lax.iota(jnp.int32, 16)
    zerov = jnp.zeros((16,), jnp.float32)

    def _start(a, c):
        row0 = (g0 + c * CHUNK_G) * 16 * VPR
        dst = xbuf.at[pl.ds((c % 2) * CROWS, CROWS)]

        @pl.when(a == 0)
        def _():
            pltpu.async_copy(x1_hbm.at[pl.ds(row0, CROWS)], dst, sem)

        @pl.when(a == 1)
        def _():
            pltpu.async_copy(x2_hbm.at[pl.ds(row0, CROWS)], dst, sem)

    def _wait_one():
        pltpu.make_async_copy(
            x1_hbm.at[pl.ds(0, CROWS)], xbuf.at[pl.ds(0, CROWS)], sem
        ).wait()

    @pl.loop(0, 2)
    def _per_array(a):
        _start(a, 0)
        pltpu.sync_copy(b_hbm.at[pl.ds(a * NGRP_PAD + g0, GPW)], idbuf)

        @pl.loop(0, GSEG)
        def _zero(r):
            for j in range(VPR):
                acc[r * VPR + j] = zerov

        for r in range(CNT_ROWS):
            cnt[r] = zerov

        cur0 = _extract(idbuf[0], 0)
        carry0 = (cur0,) + (zerov,) * VPR

        def _chunk(c, carry):
            _wait_one()

            @pl.when(c + 1 < nchunk)
            def _():
                _start(a, c + 1)

            boff = (c % 2) * CROWS

            def _group(g, carry):
                idv = idbuf[c * CHUNK_G + g]
                base = boff + g * 16 * VPR
                cur = carry[0]
                same = jnp.all(idv == cur)

                def _fast(cur, *regs):
                    new = []
                    for j in range(VPR):
                        s = regs[j]
                        for i in range(16):
                            s = s + xbuf[base + i * VPR + j]
                        new.append(s)
                    oh = jnp.where(lanes == cur % 16, jnp.float32(16.0),
                                   jnp.float32(0.0))
                    plsc.addupdate(cnt.at[cur // 16], oh)
                    return (cur, *new)

                def _slow(cur, *regs):
                    regs = list(regs)
                    for i in range(16):
                        b = _extract(idv, i)

                        def _flush(cur, *regs):
                            for j in range(VPR):
                                plsc.addupdate(acc.at[cur * VPR + j], regs[j])
                            return (b,) + (zerov,) * VPR

                        def _keep(cur, *regs):
                            return (cur, *regs)

                        cur, *regs = lax.cond(b != cur, _flush, _keep,
                                              cur, *regs)
                        for j in range(VPR):
                            regs[j] = regs[j] + xbuf[base + i * VPR + j]
                        oh = jnp.where(lanes == b % 16, jnp.float32(1.0),
                                       jnp.float32(0.0))
                        plsc.addupdate(cnt.at[b // 16], oh)
                    return (cur, *regs)

                return lax.cond(same, _fast, _slow, *carry)

            return lax.fori_loop(0, CHUNK_G, _group, carry)

        cur, *regs = lax.fori_loop(0, nchunk, _chunk, carry0)
        for j in range(VPR):
            plsc.addupdate(acc.at[cur * VPR + j], regs[j])

        pltpu.sync_copy(acc, p_out.at[a, wid])
        pltpu.sync_copy(cnt, c_out.at[a, wid])


@jax.jit
def _segsum(x1v, bv, x2v):
    mesh = plsc.VectorSubcoreMesh(
        core_axis_name="c", subcore_axis_name="s", num_cores=NC
    )
    return pl.kernel(
        _segsum_body,
        out_type=(
            jax.ShapeDtypeStruct((2, NW, ACC_ROWS, 16), jnp.float32),
            jax.ShapeDtypeStruct((2, NW, CNT_ROWS, 16), jnp.float32),
        ),
        mesh=mesh,
        compiler_params=pltpu.CompilerParams(
            use_tc_tiling_on_sc=False, needs_layout_passes=False
        ),
        scratch_types=[
            pltpu.VMEM((2 * CROWS, 16), jnp.float32),
            pltpu.VMEM((GPW, 16), jnp.int32),
            pltpu.VMEM((ACC_ROWS, 16), jnp.float32),
            pltpu.VMEM((CNT_ROWS, 16), jnp.float32),
            pltpu.SemaphoreType.DMA,
        ],
    )(x1v, bv, x2v)


def _finish_body(p_ref, c_ref, w_ref, out_ref):
    w = w_ref[...]

    def pooled(a):
        s = jnp.sum(p_ref[a], axis=0)[:G]
        cg = jnp.sum(c_ref[a], axis=0)[:G]
        c = jnp.maximum(cg, 1.0)
        m = s / c[:, None]
        return jnp.dot(m, w, preferred_element_type=jnp.float32)

    p1 = pooled(0)
    p2 = pooled(1)
    n1 = jnp.maximum(jnp.sqrt(jnp.sum(p1 * p1, axis=-1)), EPS)
    n2 = jnp.maximum(jnp.sqrt(jnp.sum(p2 * p2, axis=-1)), EPS)
    out_ref[...] = jnp.sum(p1 * p2, axis=-1) / (n1 * n2)


@jax.jit
def _finish(part, cnt, w):
    return pl.pallas_call(
        _finish_body,
        out_shape=jax.ShapeDtypeStruct((G,), jnp.float32),
    )(part, cnt, w)


def kernel(x1, batch1, x2, batch2, W):
    x1v = x1.reshape(N * VPR, 16)
    x2v = x2.reshape(N * VPR, 16)
    padb = jnp.full((NGRP_PAD * 16 - N,), G, jnp.int32)
    bv = jnp.concatenate(
        [batch1.astype(jnp.int32), padb, batch2.astype(jnp.int32), padb]
    ).reshape(2 * NGRP_PAD, 16)
    part, cnt = _segsum(x1v, bv, x2v)
    return _finish(
        part.reshape(2, NW, GSEG, D), cnt.reshape(2, NW, CNT_ROWS * 16), W
    )


# speedup vs baseline: 10.0932x; 2.0463x over previous
import jax
import jax.numpy as jnp
from jax import lax
from jax.experimental import pallas as pl
from jax.experimental.pallas import tpu as pltpu
from jax.experimental.pallas import tpu_sc as plsc

N = 100000
D = 128
G = 256
EPS = 1e-8

NC = 2
NS = 16
NW = NC * NS
VPR = D // 16
NGRP = N // 16
GPW = 200
CHUNK_G = 10
CROWS = CHUNK_G * 16
GSEG_PAD = 264


def _extract(vec, i):
    return lax.squeeze(lax.slice_in_dim(vec, i, i + 1), (0,))


def _segsum_body(x1_hbm, b1_hbm, x2_hbm, b2_hbm, p_out,
                 xbuf, idbuf, acc, sem):
    wid = lax.axis_index("s") * NC + lax.axis_index("c")
    g0 = wid * GPW
    nchunk = jnp.minimum(GPW // CHUNK_G, (NGRP - g0) // CHUNK_G)
    last = g0 + GPW > NGRP
    zerov = jnp.zeros((16,), jnp.float32)

    def _start(a, c):
        row0 = (g0 + c * CHUNK_G) * 16
        dst = xbuf.at[pl.ds((c % 2) * CROWS, CROWS)]

        @pl.when(a == 0)
        def _():
            pltpu.async_copy(x1_hbm.at[pl.ds(row0, CROWS)], dst, sem)

        @pl.when(a == 1)
        def _():
            pltpu.async_copy(x2_hbm.at[pl.ds(row0, CROWS)], dst, sem)

    def _wait_one():
        pltpu.make_async_copy(
            x1_hbm.at[pl.ds(0, CROWS)], xbuf.at[pl.ds(0, CROWS)], sem
        ).wait()

    @pl.loop(0, 2)
    def _per_array(a):
        _start(a, 0)

        i0 = g0 * 16
        for aa, b_hbm in ((0, b1_hbm), (1, b2_hbm)):
            @pl.when((a == aa) & jnp.logical_not(last))
            def _():
                pltpu.sync_copy(b_hbm.at[pl.ds(i0, GPW * 16)], idbuf)

            @pl.when((a == aa) & last)
            def _():
                pltpu.sync_copy(b_hbm.at[pl.ds(i0, 800)],
                                idbuf.at[pl.ds(0, 800)])

        @pl.loop(0, GSEG_PAD)
        def _zero(r):
            for j in range(VPR):
                acc[r, pl.ds(j * 16, 16)] = zerov

        cur0 = _extract(idbuf[pl.ds(0, 16)], 0)
        carry0 = (cur0,) + (zerov,) * VPR

        def _chunk(c, carry):
            _wait_one()

            @pl.when(c + 1 < nchunk)
            def _():
                _start(a, c + 1)

            boff = (c % 2) * CROWS

            def _group(g, carry):
                idv = idbuf[pl.ds((c * CHUNK_G + g) * 16, 16)]
                base = boff + g * 16
                cur = carry[0]
                same = jnp.all(idv == cur)

                def _fast(cur, *regs):
                    new = []
                    for j in range(VPR):
                        s = regs[j]
                        for i in range(16):
                            s = s + xbuf[base + i, pl.ds(j * 16, 16)]
                        new.append(s)
                    return (cur, *new)

                def _slow(cur, *regs):
                    regs = list(regs)
                    for i in range(16):
                        b = _extract(idv, i)

                        def _flush(cur, *regs):
                            for j in range(VPR):
                                plsc.addupdate(
                                    acc.at[cur, pl.ds(j * 16, 16)], regs[j]
                                )
                            return (b,) + (zerov,) * VPR

                        def _keep(cur, *regs):
                            return (cur, *regs)

                        cur, *regs = lax.cond(b != cur, _flush, _keep,
                                              cur, *regs)
                        for j in range(VPR):
                            regs[j] = regs[j] + xbuf[base + i,
                                                     pl.ds(j * 16, 16)]
                    return (cur, *regs)

                return lax.cond(same, _fast, _slow, *carry)

            return lax.fori_loop(0, CHUNK_G, _group, carry)

        cur, *regs = lax.fori_loop(0, nchunk, _chunk, carry0)
        for j in range(VPR):
            plsc.addupdate(acc.at[cur, pl.ds(j * 16, 16)], regs[j])

        pltpu.sync_copy(acc, p_out.at[a, wid])


@jax.jit
def _segsum(x1, b1, x2, b2):
    mesh = plsc.VectorSubcoreMesh(
        core_axis_name="c", subcore_axis_name="s", num_cores=NC
    )
    return pl.kernel(
        _segsum_body,
        out_type=jax.ShapeDtypeStruct((2, NW, GSEG_PAD, D), jnp.float32),
        mesh=mesh,
        compiler_params=pltpu.CompilerParams(
            use_tc_tiling_on_sc=False, needs_layout_passes=False
        ),
        scratch_types=[
            pltpu.VMEM((2 * CROWS, D), jnp.float32),
            pltpu.VMEM((GPW * 16,), jnp.int32),
            pltpu.VMEM((GSEG_PAD, D), jnp.float32),
            pltpu.SemaphoreType.DMA,
        ],
    )(x1, b1, x2, b2)


def _finish_body(p_ref, w_ref, out_ref):
    w = w_ref[...]

    def embed(a):
        s = jnp.sum(p_ref[a], axis=0)[:G]
        return jnp.dot(s, w, preferred_element_type=jnp.float32)

    e1 = embed(0)
    e2 = embed(1)
    n1 = jnp.maximum(jnp.sqrt(jnp.sum(e1 * e1, axis=-1)), EPS)
    n2 = jnp.maximum(jnp.sqrt(jnp.sum(e2 * e2, axis=-1)), EPS)
    out_ref[...] = jnp.sum(e1 * e2, axis=-1) / (n1 * n2)


@jax.jit
def _finish(part, w):
    return pl.pallas_call(
        _finish_body,
        out_shape=jax.ShapeDtypeStruct((G,), jnp.float32),
    )(part, w)


def kernel(x1, batch1, x2, batch2, W):
    part = _segsum(x1, batch1.astype(jnp.int32), x2,
                   batch2.astype(jnp.int32))
    return _finish(part, W)
